# Initial kernel scaffold; baseline (speedup 1.0000x reference)
import jax
import jax.numpy as jnp
from jax.experimental import pallas as pl


def kernel(edge_index, now_user_degree, now_item_degree, old_user_degree, old_item_degree, old_emb0, old_emb1, user_table, item_table, conv_w):
    raise NotImplementedError("write your pallas kernel here")


# speedup vs baseline: 3.3232x; 3.3232x over previous
import functools

import jax
import jax.numpy as jnp
from jax import lax
from jax.experimental import pallas as pl
from jax.experimental.pallas import tpu as pltpu
from jax.experimental.pallas import tpu_sc as plsc

D = 64
HALF = 32
N_PAD = 51200
CHUNK = 128
NS = 16
NC = 2
ZR = 128
BR = 512


def _prep_body(cw_ref, tot_ref, dn_ref, do_ref, old_ref, s3_ref, tv_ref):
    dn = dn_ref[...]
    do = do_ref[...]
    inv = 1.0 / (jnp.sqrt(dn + do) + 1e-9)
    scaled = inv * tot_ref[...]
    w10 = cw_ref[1, 0]
    tv_ref[...] = (w10 * jnp.sqrt(do) * inv) * old_ref[...]
    s3_ref[0] = scaled[:, :HALF]
    s3_ref[1] = scaled[:, HALF:]


def _post_body(cw_ref, p3_ref, tv_ref, tot_ref, dn_ref, do_ref, out_ref):
    dn = dn_ref[...]
    do = do_ref[...]
    inv = 1.0 / (jnp.sqrt(dn + do) + 1e-9)
    w11 = cw_ref[1, 1]
    p = jnp.concatenate([p3_ref[0], p3_ref[1]], axis=1)
    last = tv_ref[...] + (w11 * inv) * p
    nrm = jnp.maximum(
        jnp.sqrt(jnp.sum(last * last, axis=-1, keepdims=True)), 1e-12)
    out_ref[...] = last / nrm + tot_ref[...]


def _sc_spmm(e_pad):
    e_per_tile = e_pad // NS
    n_chunks = e_per_tile // CHUNK
    rows_per_tile = N_PAD // NS
    n_zr = rows_per_tile // ZR
    mesh = plsc.VectorSubcoreMesh(core_axis_name="c", subcore_axis_name="s")

    @functools.partial(
        pl.kernel,
        out_type=jax.ShapeDtypeStruct((NC, N_PAD, HALF), jnp.float32),
        mesh=mesh,
        compiler_params=pltpu.CompilerParams(use_tc_tiling_on_sc=False),
        scratch_types=[
            pltpu.VMEM((CHUNK,), jnp.int32),
            pltpu.VMEM((CHUNK,), jnp.int32),
            pltpu.VMEM((CHUNK, HALF), jnp.float32),
            pltpu.VMEM((ZR, HALF), jnp.float32),
            pltpu.VMEM_SHARED((N_PAD, HALF), jnp.float32),
            pltpu.SemaphoreType.DMA,
        ],
    )
    def spmm(src_hbm, dst_hbm, s3_hbm, out_hbm,
             src_v, dst_v, rows_v, zbuf, acc, sem):
        c = lax.axis_index("c")
        s = lax.axis_index("s")

        def zrow(i, carry):
            z = jnp.zeros((16,), jnp.float32)
            zbuf[i, pl.ds(0, 16)] = z
            zbuf[i, pl.ds(16, 16)] = z
            return carry
        lax.fori_loop(0, ZR, zrow, 0)

        rbase = s * rows_per_tile

        def zcopy(i, carry):
            pltpu.sync_copy(zbuf, acc.at[pl.ds(rbase + i * ZR, ZR)])
            return carry
        lax.fori_loop(0, n_zr, zcopy, 0)
        plsc.subcore_barrier()

        ebase = s * e_per_tile

        def echunk(j, carry):
            off = ebase + j * CHUNK
            pltpu.sync_copy(src_hbm.at[pl.ds(off, CHUNK)], src_v)
            pltpu.sync_copy(dst_hbm.at[pl.ds(off, CHUNK)], dst_v)
            pltpu.async_copy(s3_hbm.at[c].at[src_v], rows_v, sem).wait()
            pltpu.sync_copy(rows_v, acc.at[dst_v], add=True)
            return carry
        lax.fori_loop(0, n_chunks, echunk, 0)
        plsc.subcore_barrier()

        def wcopy(i, carry):
            r = rbase + i * ZR
            pltpu.sync_copy(acc.at[pl.ds(r, ZR)], zbuf)
            pltpu.sync_copy(zbuf, out_hbm.at[c].at[pl.ds(r, ZR)])
            return carry
        lax.fori_loop(0, n_zr, wcopy, 0)

    return spmm


def kernel(edge_index, now_user_degree, now_item_degree, old_user_degree,
           old_item_degree, old_emb0, old_emb1, user_table, item_table,
           conv_w):
    n = user_table.shape[0] + item_table.shape[0]
    e = edge_index.shape[1]
    e_pad = -(-e // (NS * CHUNK)) * (NS * CHUNK)

    total = jnp.concatenate([user_table, item_table], axis=0)
    deg_new = jnp.concatenate([now_user_degree, now_item_degree], axis=0)
    deg_old = jnp.concatenate([old_user_degree, old_item_degree], axis=0)
    pad_n = N_PAD - n
    totp = jnp.pad(total, ((0, pad_n), (0, 0)))
    dnp = jnp.pad(deg_new, ((0, pad_n), (0, 0)))
    dop = jnp.pad(deg_old, ((0, pad_n), (0, 0)))
    oldp = jnp.pad(old_emb1, ((0, pad_n), (0, 0)))
    ei = edge_index.astype(jnp.int32)
    pad_e = e_pad - e
    src = jnp.concatenate([ei[1], jnp.zeros((pad_e,), jnp.int32)])
    dst = jnp.concatenate([ei[0], jnp.full((pad_e,), N_PAD - 1, jnp.int32)])

    grid = (N_PAD // BR,)
    row_spec = pl.BlockSpec((BR, D), lambda i: (i, 0))
    col_spec = pl.BlockSpec((BR, 1), lambda i: (i, 0))
    s3_spec = pl.BlockSpec((NC, BR, HALF), lambda i: (0, i, 0))
    cw_spec = pl.BlockSpec(memory_space=pltpu.SMEM)

    scaled3, tvec = pl.pallas_call(
        _prep_body,
        grid=grid,
        in_specs=[cw_spec, row_spec, col_spec, col_spec, row_spec],
        out_specs=[s3_spec, row_spec],
        out_shape=[
            jax.ShapeDtypeStruct((NC, N_PAD, HALF), jnp.float32),
            jax.ShapeDtypeStruct((N_PAD, D), jnp.float32),
        ],
    )(conv_w, totp, dnp, dop, oldp)

    p3 = _sc_spmm(e_pad)(src, dst, scaled3)

    outp = pl.pallas_call(
        _post_body,
        grid=grid,
        in_specs=[cw_spec, s3_spec, row_spec, row_spec, col_spec, col_spec],
        out_specs=row_spec,
        out_shape=jax.ShapeDtypeStruct((N_PAD, D), jnp.float32),
    )(conv_w, p3, tvec, totp, dnp, dop)

    return outp[:n]


# speedup vs baseline: 4.2566x; 1.2809x over previous
import functools

import jax
import jax.numpy as jnp
from jax import lax
from jax.experimental import pallas as pl
from jax.experimental.pallas import tpu as pltpu
from jax.experimental.pallas import tpu_sc as plsc

D = 64
HALF = 32
N_PAD = 51200
CHUNK = 128
NS = 16
NC = 2
ZR = 128
BR = 512


def _prep_body(cw_ref, tot_ref, dn_ref, do_ref, old_ref, s3_ref, tv_ref):
    dn = dn_ref[...]
    do = do_ref[...]
    inv = 1.0 / (jnp.sqrt(dn + do) + 1e-9)
    scaled = inv * tot_ref[...]
    w10 = cw_ref[1, 0]
    tv_ref[...] = (w10 * jnp.sqrt(do) * inv) * old_ref[...]
    s3_ref[0] = scaled[:, :HALF]
    s3_ref[1] = scaled[:, HALF:]


def _post_body(cw_ref, p3_ref, tv_ref, tot_ref, dn_ref, do_ref, out_ref):
    dn = dn_ref[...]
    do = do_ref[...]
    inv = 1.0 / (jnp.sqrt(dn + do) + 1e-9)
    w11 = cw_ref[1, 1]
    p = jnp.concatenate([p3_ref[0], p3_ref[1]], axis=1)
    last = tv_ref[...] + (w11 * inv) * p
    nrm = jnp.maximum(
        jnp.sqrt(jnp.sum(last * last, axis=-1, keepdims=True)), 1e-12)
    out_ref[...] = last / nrm + tot_ref[...]


IB = 16


def _sc_spmm(e_pad):
    e_per_tile = e_pad // NS
    n_chunks = e_per_tile // CHUNK
    nb = n_chunks // IB
    rows_per_tile = N_PAD // NS
    n_zr = rows_per_tile // ZR
    mesh = plsc.VectorSubcoreMesh(core_axis_name="c", subcore_axis_name="s")

    @functools.partial(
        pl.kernel,
        out_type=jax.ShapeDtypeStruct((NC, N_PAD, HALF), jnp.float32),
        mesh=mesh,
        compiler_params=pltpu.CompilerParams(use_tc_tiling_on_sc=False),
        scratch_types=[
            pltpu.VMEM((2, IB, CHUNK), jnp.int32),
            pltpu.VMEM((2, IB, CHUNK), jnp.int32),
            pltpu.VMEM((2, CHUNK, HALF), jnp.float32),
            pltpu.VMEM_SHARED((N_PAD, HALF), jnp.float32),
            pltpu.SemaphoreType.DMA,
            pltpu.SemaphoreType.DMA,
        ],
    )
    def spmm(src_hbm, dst_hbm, s3_hbm, out_hbm,
             sibuf, dibuf, ring, acc, gsem, isem):
        c = lax.axis_index("c")
        s = lax.axis_index("s")
        s3c = s3_hbm.at[c]
        cbase = s * n_chunks

        def zrow(i, carry):
            z = jnp.zeros((16,), jnp.float32)
            ring[0, i, pl.ds(0, 16)] = z
            ring[0, i, pl.ds(16, 16)] = z
            return carry
        lax.fori_loop(0, CHUNK, zrow, 0)
        rbase = s * rows_per_tile

        def zcopy(i, carry):
            pltpu.sync_copy(ring.at[0], acc.at[pl.ds(rbase + i * ZR, ZR)])
            return carry
        lax.fori_loop(0, n_zr, zcopy, 0)
        plsc.subcore_barrier()

        pltpu.sync_copy(src_hbm.at[pl.ds(cbase, IB)], sibuf.at[0])
        pltpu.sync_copy(dst_hbm.at[pl.ds(cbase, IB)], dibuf.at[0])
        pltpu.async_copy(s3c.at[sibuf.at[0].at[0]], ring.at[0], gsem)

        def block(b, carry):
            pb = lax.rem(b, 2)
            nxt = cbase + (b + 1) * IB

            @pl.when(b < nb - 1)
            def _():
                pltpu.async_copy(
                    src_hbm.at[pl.ds(nxt, IB)], sibuf.at[1 - pb], isem)
                pltpu.async_copy(
                    dst_hbm.at[pl.ds(nxt, IB)], dibuf.at[1 - pb], isem)

            sib = sibuf.at[pb]
            dib = dibuf.at[pb]
            for k in range(IB):
                slot = k % 2
                pltpu.make_async_copy(
                    s3c.at[sib.at[k]], ring.at[slot], gsem).wait()
                if k < IB - 1:
                    pltpu.async_copy(
                        s3c.at[sib.at[k + 1]], ring.at[1 - slot], gsem)
                else:
                    @pl.when(b < nb - 1)
                    def _():
                        pltpu.make_async_copy(
                            src_hbm.at[pl.ds(nxt, IB)], sibuf.at[1 - pb],
                            isem).wait()
                        pltpu.make_async_copy(
                            dst_hbm.at[pl.ds(nxt, IB)], dibuf.at[1 - pb],
                            isem).wait()
                        pltpu.async_copy(
                            s3c.at[sibuf.at[1 - pb].at[0]], ring.at[1 - slot],
                            gsem)
                pltpu.sync_copy(ring.at[slot], acc.at[dib.at[k]], add=True)
            return carry
        lax.fori_loop(0, nb, block, 0)
        plsc.subcore_barrier()

        def wcopy(i, carry):
            r = rbase + i * ZR
            pltpu.sync_copy(acc.at[pl.ds(r, ZR)], ring.at[0])
            pltpu.sync_copy(ring.at[0], out_hbm.at[c].at[pl.ds(r, ZR)])
            return carry
        lax.fori_loop(0, n_zr, wcopy, 0)

    return spmm


def kernel(edge_index, now_user_degree, now_item_degree, old_user_degree,
           old_item_degree, old_emb0, old_emb1, user_table, item_table,
           conv_w):
    n = user_table.shape[0] + item_table.shape[0]
    e = edge_index.shape[1]
    e_pad = -(-e // (NS * CHUNK * IB)) * (NS * CHUNK * IB)

    total = jnp.concatenate([user_table, item_table], axis=0)
    deg_new = jnp.concatenate([now_user_degree, now_item_degree], axis=0)
    deg_old = jnp.concatenate([old_user_degree, old_item_degree], axis=0)
    pad_n = N_PAD - n
    totp = jnp.pad(total, ((0, pad_n), (0, 0)))
    dnp = jnp.pad(deg_new, ((0, pad_n), (0, 0)))
    dop = jnp.pad(deg_old, ((0, pad_n), (0, 0)))
    oldp = jnp.pad(old_emb1, ((0, pad_n), (0, 0)))
    ei = edge_index.astype(jnp.int32)
    pad_e = e_pad - e
    src = jnp.concatenate(
        [ei[1], jnp.zeros((pad_e,), jnp.int32)]).reshape(-1, CHUNK)
    dst = jnp.concatenate(
        [ei[0], jnp.full((pad_e,), N_PAD - 1, jnp.int32)]).reshape(-1, CHUNK)

    grid = (N_PAD // BR,)
    row_spec = pl.BlockSpec((BR, D), lambda i: (i, 0))
    col_spec = pl.BlockSpec((BR, 1), lambda i: (i, 0))
    s3_spec = pl.BlockSpec((NC, BR, HALF), lambda i: (0, i, 0))
    cw_spec = pl.BlockSpec(memory_space=pltpu.SMEM)

    scaled3, tvec = pl.pallas_call(
        _prep_body,
        grid=grid,
        in_specs=[cw_spec, row_spec, col_spec, col_spec, row_spec],
        out_specs=[s3_spec, row_spec],
        out_shape=[
            jax.ShapeDtypeStruct((NC, N_PAD, HALF), jnp.float32),
            jax.ShapeDtypeStruct((N_PAD, D), jnp.float32),
        ],
    )(conv_w, totp, dnp, dop, oldp)

    p3 = _sc_spmm(e_pad)(src, dst, scaled3)

    outp = pl.pallas_call(
        _post_body,
        grid=grid,
        in_specs=[cw_spec, s3_spec, row_spec, row_spec, col_spec, col_spec],
        out_specs=row_spec,
        out_shape=jax.ShapeDtypeStruct((N_PAD, D), jnp.float32),
    )(conv_w, p3, tvec, totp, dnp, dop)

    return outp[:n]


# speedup vs baseline: 5.1324x; 1.2058x over previous
import functools

import jax
import jax.numpy as jnp
from jax import lax
from jax.experimental import pallas as pl
from jax.experimental.pallas import tpu as pltpu
from jax.experimental.pallas import tpu_sc as plsc

D = 64
HALF = 32
N_PAD = 51200
CHUNK = 128
NS = 16
NC = 2
ZR = 128
BR = 512


def _prep_body(cw_ref, tot_ref, dn_ref, do_ref, old_ref, s3_ref, tv_ref):
    dn = dn_ref[...]
    do = do_ref[...]
    inv = 1.0 / (jnp.sqrt(dn + do) + 1e-9)
    scaled = inv * tot_ref[...]
    w10 = cw_ref[1, 0]
    tv_ref[...] = (w10 * jnp.sqrt(do) * inv) * old_ref[...]
    s3_ref[0] = scaled[:, :HALF]
    s3_ref[1] = scaled[:, HALF:]


def _post_body(cw_ref, p3_ref, tv_ref, tot_ref, dn_ref, do_ref, out_ref):
    dn = dn_ref[...]
    do = do_ref[...]
    inv = 1.0 / (jnp.sqrt(dn + do) + 1e-9)
    w11 = cw_ref[1, 1]
    p = jnp.concatenate([p3_ref[0], p3_ref[1]], axis=1)
    last = tv_ref[...] + (w11 * inv) * p
    nrm = jnp.maximum(
        jnp.sqrt(jnp.sum(last * last, axis=-1, keepdims=True)), 1e-12)
    out_ref[...] = last / nrm + tot_ref[...]


IB = 16


def _sc_spmm(e_pad):
    e_per_tile = e_pad // NS
    n_chunks = e_per_tile // CHUNK
    nb = n_chunks // IB
    rows_per_tile = N_PAD // NS
    n_zr = rows_per_tile // ZR
    mesh = plsc.VectorSubcoreMesh(core_axis_name="c", subcore_axis_name="s")

    @functools.partial(
        pl.kernel,
        out_type=jax.ShapeDtypeStruct((NC, N_PAD, HALF), jnp.float32),
        mesh=mesh,
        compiler_params=pltpu.CompilerParams(use_tc_tiling_on_sc=False),
        scratch_types=[
            pltpu.VMEM((2, IB, CHUNK), jnp.int32),
            pltpu.VMEM((2, IB, CHUNK), jnp.int32),
            pltpu.VMEM((4, CHUNK, HALF), jnp.float32),
            pltpu.VMEM_SHARED((N_PAD, HALF), jnp.float32),
            pltpu.SemaphoreType.DMA,
            pltpu.SemaphoreType.DMA,
            pltpu.SemaphoreType.DMA,
        ],
    )
    def spmm(src_hbm, dst_hbm, s3_hbm, out_hbm,
             sibuf, dibuf, ring, acc, gsem, isem, ssem):
        c = lax.axis_index("c")
        s = lax.axis_index("s")
        s3c = s3_hbm.at[c]
        cbase = s * n_chunks

        def zrow(i, carry):
            z = jnp.zeros((16,), jnp.float32)
            ring[0, i, pl.ds(0, 16)] = z
            ring[0, i, pl.ds(16, 16)] = z
            return carry
        lax.fori_loop(0, CHUNK, zrow, 0)
        rbase = s * rows_per_tile

        def zcopy(i, carry):
            pltpu.sync_copy(ring.at[0], acc.at[pl.ds(rbase + i * ZR, ZR)])
            return carry
        lax.fori_loop(0, n_zr, zcopy, 0)
        plsc.subcore_barrier()

        pltpu.sync_copy(src_hbm.at[pl.ds(cbase, IB)], sibuf.at[0])
        pltpu.sync_copy(dst_hbm.at[pl.ds(cbase, IB)], dibuf.at[0])
        pltpu.async_copy(s3c.at[sibuf.at[0].at[0]], ring.at[0], gsem)
        pltpu.async_copy(s3c.at[sibuf.at[0].at[1]], ring.at[1], gsem)

        def block(b, carry):
            pb = lax.rem(b, 2)
            nxt = cbase + (b + 1) * IB

            @pl.when(b < nb - 1)
            def _():
                pltpu.async_copy(
                    src_hbm.at[pl.ds(nxt, IB)], sibuf.at[1 - pb], isem)
                pltpu.async_copy(
                    dst_hbm.at[pl.ds(nxt, IB)], dibuf.at[1 - pb], isem)

            sib = sibuf.at[pb]
            dib = dibuf.at[pb]
            sibn = sibuf.at[1 - pb]
            for k in range(IB):
                slot = k % 4
                s2 = (k + 2) % 4
                if k >= 2:
                    pltpu.make_async_copy(
                        ring.at[s2], acc.at[dib.at[k - 2]], ssem).wait()
                    if k < IB - 2:
                        pltpu.async_copy(
                            s3c.at[sib.at[k + 2]], ring.at[s2], gsem)
                    else:
                        @pl.when(b < nb - 1)
                        def _():
                            if k == IB - 2:
                                pltpu.make_async_copy(
                                    src_hbm.at[pl.ds(nxt, IB)],
                                    sibuf.at[1 - pb], isem).wait()
                                pltpu.make_async_copy(
                                    dst_hbm.at[pl.ds(nxt, IB)],
                                    dibuf.at[1 - pb], isem).wait()
                            pltpu.async_copy(
                                s3c.at[sibn.at[k - (IB - 2)]], ring.at[s2],
                                gsem)
                else:
                    @pl.when(b > 0)
                    def _():
                        pltpu.make_async_copy(
                            ring.at[s2], acc.at[dib.at[k]], ssem).wait()
                    pltpu.async_copy(
                        s3c.at[sib.at[k + 2]], ring.at[s2], gsem)
                pltpu.make_async_copy(
                    s3c.at[sib.at[k]], ring.at[slot], gsem).wait()
                pltpu.async_copy(
                    ring.at[slot], acc.at[dib.at[k]], ssem, add=True)
            return carry
        lax.fori_loop(0, nb, block, 0)
        pltpu.make_async_copy(
            ring.at[2], acc.at[dibuf.at[0].at[0]], ssem).wait()
        pltpu.make_async_copy(
            ring.at[3], acc.at[dibuf.at[0].at[1]], ssem).wait()
        plsc.subcore_barrier()

        def wcopy(i, carry):
            r = rbase + i * ZR
            pltpu.sync_copy(acc.at[pl.ds(r, ZR)], ring.at[0])
            pltpu.sync_copy(ring.at[0], out_hbm.at[c].at[pl.ds(r, ZR)])
            return carry
        lax.fori_loop(0, n_zr, wcopy, 0)

    return spmm


def kernel(edge_index, now_user_degree, now_item_degree, old_user_degree,
           old_item_degree, old_emb0, old_emb1, user_table, item_table,
           conv_w):
    n = user_table.shape[0] + item_table.shape[0]
    e = edge_index.shape[1]
    e_pad = -(-e // (NS * CHUNK * IB)) * (NS * CHUNK * IB)

    total = jnp.concatenate([user_table, item_table], axis=0)
    deg_new = jnp.concatenate([now_user_degree, now_item_degree], axis=0)
    deg_old = jnp.concatenate([old_user_degree, old_item_degree], axis=0)
    pad_n = N_PAD - n
    totp = jnp.pad(total, ((0, pad_n), (0, 0)))
    dnp = jnp.pad(deg_new, ((0, pad_n), (0, 0)))
    dop = jnp.pad(deg_old, ((0, pad_n), (0, 0)))
    oldp = jnp.pad(old_emb1, ((0, pad_n), (0, 0)))
    ei = edge_index.astype(jnp.int32)
    pad_e = e_pad - e
    src = jnp.concatenate(
        [ei[1], jnp.zeros((pad_e,), jnp.int32)]).reshape(-1, CHUNK)
    dst = jnp.concatenate(
        [ei[0], jnp.full((pad_e,), N_PAD - 1, jnp.int32)]).reshape(-1, CHUNK)

    grid = (N_PAD // BR,)
    row_spec = pl.BlockSpec((BR, D), lambda i: (i, 0))
    col_spec = pl.BlockSpec((BR, 1), lambda i: (i, 0))
    s3_spec = pl.BlockSpec((NC, BR, HALF), lambda i: (0, i, 0))
    cw_spec = pl.BlockSpec(memory_space=pltpu.SMEM)

    scaled3, tvec = pl.pallas_call(
        _prep_body,
        grid=grid,
        in_specs=[cw_spec, row_spec, col_spec, col_spec, row_spec],
        out_specs=[s3_spec, row_spec],
        out_shape=[
            jax.ShapeDtypeStruct((NC, N_PAD, HALF), jnp.float32),
            jax.ShapeDtypeStruct((N_PAD, D), jnp.float32),
        ],
    )(conv_w, totp, dnp, dop, oldp)

    p3 = _sc_spmm(e_pad)(src, dst, scaled3)

    outp = pl.pallas_call(
        _post_body,
        grid=grid,
        in_specs=[cw_spec, s3_spec, row_spec, row_spec, col_spec, col_spec],
        out_specs=row_spec,
        out_shape=jax.ShapeDtypeStruct((N_PAD, D), jnp.float32),
    )(conv_w, p3, tvec, totp, dnp, dop)

    return outp[:n]


# speedup vs baseline: 5.4700x; 1.0658x over previous
import functools

import jax
import jax.numpy as jnp
from jax import lax
from jax.experimental import pallas as pl
from jax.experimental.pallas import tpu as pltpu
from jax.experimental.pallas import tpu_sc as plsc

D = 64
HALF = 32
N_PAD = 51200
CHUNK = 128
NS = 16
NC = 2
ZR = 128
BR = 512
IB = 16


def _prep_body(tot_ref, dn_ref, do_ref, s3_ref):
    inv = 1.0 / (jnp.sqrt(dn_ref[...] + do_ref[...]) + 1e-9)
    scaled = inv * tot_ref[...]
    s3_ref[0] = scaled[:, :HALF]
    s3_ref[1] = scaled[:, HALF:]


def _post_body(cw_ref, p3_ref, tot_ref, old_ref, dn_ref, do_ref, out_ref):
    do = do_ref[...]
    inv = 1.0 / (jnp.sqrt(dn_ref[...] + do) + 1e-9)
    w10 = cw_ref[1, 0]
    w11 = cw_ref[1, 1]
    p = jnp.concatenate([p3_ref[0], p3_ref[1]], axis=1)
    last = (w10 * jnp.sqrt(do) * inv) * old_ref[...] + (w11 * inv) * p
    nrm = jnp.maximum(
        jnp.sqrt(jnp.sum(last * last, axis=-1, keepdims=True)), 1e-12)
    out_ref[...] = last / nrm + tot_ref[...]


def _sc_spmm(n_nodes, e_pad):
    e_per_tile = e_pad // NS
    n_chunks = e_per_tile // CHUNK
    nb = n_chunks // IB
    rows_per_tile = N_PAD // NS
    n_zr = rows_per_tile // ZR
    mesh = plsc.VectorSubcoreMesh(core_axis_name="c", subcore_axis_name="s")

    @functools.partial(
        pl.kernel,
        out_type=jax.ShapeDtypeStruct((NC, N_PAD, HALF), jnp.float32),
        mesh=mesh,
        compiler_params=pltpu.CompilerParams(use_tc_tiling_on_sc=False),
        scratch_types=[
            pltpu.VMEM((2, IB * CHUNK), jnp.int32),
            pltpu.VMEM((2, IB, CHUNK), jnp.int32),
            pltpu.VMEM((4, CHUNK, HALF), jnp.float32),
            pltpu.VMEM_SHARED((N_PAD, HALF), jnp.float32),
            pltpu.SemaphoreType.DMA,
            pltpu.SemaphoreType.DMA,
            pltpu.SemaphoreType.DMA,
        ],
    )
    def spmm(src_hbm, dst_hbm, s3_hbm, out_hbm,
             sibuf, dibuf, ring, acc, gsem, isem, ssem):
        c = lax.axis_index("c")
        s = lax.axis_index("s")
        s3c = s3_hbm.at[c]
        cbase = s * n_chunks
        ebase = s * e_per_tile

        def zrow(i, carry):
            z = jnp.zeros((16,), jnp.float32)
            ring[0, i, pl.ds(0, 16)] = z
            ring[0, i, pl.ds(16, 16)] = z
            return carry
        lax.fori_loop(0, CHUNK, zrow, 0)
        rbase = s * rows_per_tile

        def zcopy(i, carry):
            pltpu.sync_copy(ring.at[0], acc.at[pl.ds(rbase + i * ZR, ZR)])
            return carry
        lax.fori_loop(0, n_zr, zcopy, 0)
        plsc.subcore_barrier()

        pltpu.sync_copy(src_hbm.at[pl.ds(ebase, IB * CHUNK)], sibuf.at[0])
        pltpu.sync_copy(dst_hbm.at[pl.ds(cbase, IB)], dibuf.at[0])
        sib0 = sibuf.at[0]
        pltpu.async_copy(s3c.at[sib0.at[pl.ds(0, CHUNK)]], ring.at[0], gsem)
        pltpu.async_copy(s3c.at[sib0.at[pl.ds(CHUNK, CHUNK)]], ring.at[1],
                         gsem)

        def block(b, carry):
            pb = lax.rem(b, 2)
            nxt_e = ebase + (b + 1) * IB * CHUNK
            nxt_c = cbase + (b + 1) * IB

            @pl.when(b < nb - 1)
            def _():
                pltpu.async_copy(
                    src_hbm.at[pl.ds(nxt_e, IB * CHUNK)], sibuf.at[1 - pb],
                    isem)
                pltpu.async_copy(
                    dst_hbm.at[pl.ds(nxt_c, IB)], dibuf.at[1 - pb], isem)

            sib = sibuf.at[pb]
            dib = dibuf.at[pb]
            sibn = sibuf.at[1 - pb]
            for k in range(IB):
                slot = k % 4
                s2 = (k + 2) % 4
                if k >= 2:
                    pltpu.make_async_copy(
                        ring.at[s2], acc.at[dib.at[k - 2]], ssem).wait()
                    if k < IB - 2:
                        pltpu.async_copy(
                            s3c.at[sib.at[pl.ds((k + 2) * CHUNK, CHUNK)]],
                            ring.at[s2], gsem)
                    else:
                        @pl.when(b < nb - 1)
                        def _():
                            if k == IB - 2:
                                pltpu.make_async_copy(
                                    src_hbm.at[pl.ds(nxt_e, IB * CHUNK)],
                                    sibuf.at[1 - pb], isem).wait()
                                pltpu.make_async_copy(
                                    dst_hbm.at[pl.ds(nxt_c, IB)],
                                    dibuf.at[1 - pb], isem).wait()
                            pltpu.async_copy(
                                s3c.at[sibn.at[
                                    pl.ds((k - (IB - 2)) * CHUNK, CHUNK)]],
                                ring.at[s2], gsem)
                else:
                    @pl.when(b > 0)
                    def _():
                        pltpu.make_async_copy(
                            ring.at[s2], acc.at[dib.at[k]], ssem).wait()
                    pltpu.async_copy(
                        s3c.at[sib.at[pl.ds((k + 2) * CHUNK, CHUNK)]],
                        ring.at[s2], gsem)
                pltpu.make_async_copy(
                    s3c.at[sib.at[pl.ds(k * CHUNK, CHUNK)]], ring.at[slot],
                    gsem).wait()
                pltpu.async_copy(
                    ring.at[slot], acc.at[dib.at[k]], ssem, add=True)
            return carry
        lax.fori_loop(0, nb, block, 0)
        pltpu.make_async_copy(
            ring.at[2], acc.at[dibuf.at[0].at[0]], ssem).wait()
        pltpu.make_async_copy(
            ring.at[3], acc.at[dibuf.at[0].at[1]], ssem).wait()
        plsc.subcore_barrier()

        def wcopy(i, carry):
            r = rbase + i * ZR
            pltpu.sync_copy(acc.at[pl.ds(r, ZR)], ring.at[0])
            pltpu.sync_copy(ring.at[0], out_hbm.at[c].at[pl.ds(r, ZR)])
            return carry
        lax.fori_loop(0, n_zr, wcopy, 0)

    return spmm


def kernel(edge_index, now_user_degree, now_item_degree, old_user_degree,
           old_item_degree, old_emb0, old_emb1, user_table, item_table,
           conv_w):
    n = user_table.shape[0] + item_table.shape[0]
    e = edge_index.shape[1]
    e_pad = -(-e // (NS * CHUNK * IB)) * (NS * CHUNK * IB)

    total = jnp.concatenate([user_table, item_table], axis=0)
    deg_new = jnp.concatenate([now_user_degree, now_item_degree], axis=0)
    deg_old = jnp.concatenate([old_user_degree, old_item_degree], axis=0)
    ei = edge_index.astype(jnp.int32)
    pad_e = e_pad - e
    src = jnp.concatenate([ei[1], jnp.zeros((pad_e,), jnp.int32)])
    dst = jnp.concatenate(
        [ei[0], jnp.full((pad_e,), N_PAD - 1, jnp.int32)]).reshape(-1, CHUNK)

    ngrid = (-(-n // BR),)
    row_spec = pl.BlockSpec((BR, D), lambda i: (i, 0))
    col_spec = pl.BlockSpec((BR, 1), lambda i: (i, 0))
    s3_spec = pl.BlockSpec((NC, BR, HALF), lambda i: (0, i, 0))
    cw_spec = pl.BlockSpec(memory_space=pltpu.SMEM)

    scaled3 = pl.pallas_call(
        _prep_body,
        grid=ngrid,
        in_specs=[row_spec, col_spec, col_spec],
        out_specs=s3_spec,
        out_shape=jax.ShapeDtypeStruct((NC, n, HALF), jnp.float32),
    )(total, deg_new, deg_old)

    p3 = _sc_spmm(n, e_pad)(src, dst, scaled3)

    out = pl.pallas_call(
        _post_body,
        grid=ngrid,
        in_specs=[cw_spec, s3_spec, row_spec, row_spec, col_spec, col_spec],
        out_specs=row_spec,
        out_shape=jax.ShapeDtypeStruct((n, D), jnp.float32),
    )(conv_w, p3, total, old_emb1, deg_new, deg_old)

    return out
